# Initial kernel scaffold; baseline (speedup 1.0000x reference)
import jax
import jax.numpy as jnp
from jax.experimental import pallas as pl


def kernel(seq, token_table, pos_table):
    raise NotImplementedError("write your pallas kernel here")


# speedup vs baseline: 1.9293x; 1.9293x over previous
import functools

import jax
import jax.numpy as jnp
from jax import lax
from jax.experimental import pallas as pl
from jax.experimental.pallas import tpu as pltpu
from jax.experimental.pallas import tpu_sc as plsc

NC = 2
NS = 16
NW = NC * NS
LANES = 16

CHUNK = 128


def _seq_embed_call(n_rows, depth, pos_len):
    rows_per_w = n_rows // NW
    n_chunks = rows_per_w // CHUNK
    vregs_per_row = depth // LANES
    mesh = plsc.VectorSubcoreMesh(core_axis_name="c", subcore_axis_name="s")

    @functools.partial(
        pl.kernel,
        mesh=mesh,
        out_type=jax.ShapeDtypeStruct((n_rows, depth), jnp.float32),
        scratch_types=[
            pltpu.VMEM((rows_per_w,), jnp.int32),
            pltpu.VMEM((pos_len + CHUNK, depth), jnp.float32),
            pltpu.VMEM((CHUNK, depth), jnp.float32),
            pltpu.SemaphoreType.DMA,
        ],
    )
    def run(seq_hbm, tok_hbm, pos_hbm, out_hbm, idx_v, pos_v, rows_v, sem):
        wid = lax.axis_index("s") * NC + lax.axis_index("c")
        base = wid * rows_per_w
        pltpu.sync_copy(seq_hbm.at[pl.ds(base, rows_per_w)], idx_v)
        pltpu.sync_copy(pos_hbm, pos_v)

        def chunk_body(i, carry):
            row0 = i * CHUNK
            pltpu.async_copy(
                tok_hbm.at[idx_v.at[pl.ds(row0, CHUNK)]], rows_v, sem
            ).wait()
            phase = lax.rem(row0, pos_len)

            def row_body(r, c):
                for k in range(vregs_per_row):
                    sl = pl.ds(k * LANES, LANES)
                    rows_v[r, sl] = rows_v[r, sl] + pos_v[phase + r, sl]
                return c

            lax.fori_loop(0, CHUNK, row_body, 0)
            pltpu.async_copy(
                rows_v, out_hbm.at[pl.ds(base + row0, CHUNK), :], sem
            ).wait()
            return carry

        lax.fori_loop(0, n_chunks, chunk_body, 0)

    return run


def kernel(seq, token_table, pos_table):
    batch, seq_len = seq.shape
    vocab, depth = token_table.shape
    pos_len = pos_table.shape[0]
    n_rows = batch * seq_len
    assert n_rows % (NW * CHUNK) == 0 and depth % LANES == 0
    assert (NW * (n_rows // NW)) % pos_len == 0

    seq_flat = seq.reshape(n_rows).astype(jnp.int32)
    pos_dup = jnp.concatenate([pos_table, pos_table[:CHUNK]], axis=0)

    out = _seq_embed_call(n_rows, depth, pos_len)(seq_flat, token_table, pos_dup)
    return out.reshape(batch, seq_len, depth)


# speedup vs baseline: 9.3387x; 4.8404x over previous
import functools

import jax
import jax.numpy as jnp
from jax import lax
from jax.experimental import pallas as pl
from jax.experimental.pallas import tpu as pltpu
from jax.experimental.pallas import tpu_sc as plsc

NC = 2
NS = 16
NW = NC * NS
LANES = 16
NBUF = 4


def _seq_embed_call(batch, seq_len, depth):
    bpw = batch // NW
    nvr = depth // LANES
    mesh = plsc.VectorSubcoreMesh(core_axis_name="c", subcore_axis_name="s")

    @functools.partial(
        pl.kernel,
        mesh=mesh,
        out_type=jax.ShapeDtypeStruct((batch, seq_len, depth), jnp.float32),
        scratch_types=[
            pltpu.VMEM((seq_len, bpw), jnp.int32),
            pltpu.VMEM((seq_len, depth), jnp.float32),
        ]
        + [pltpu.VMEM((bpw, depth), jnp.float32) for _ in range(NBUF)]
        + [pltpu.SemaphoreType.DMA for _ in range(NBUF)],
    )
    def run(seq_hbm, tok_hbm, pos_hbm, out_hbm, idx_v, pos_v, *rest):
        bufs, sems = rest[:NBUF], rest[NBUF:]
        wid = lax.axis_index("s") * NC + lax.axis_index("c")
        b0 = wid * bpw
        pltpu.sync_copy(seq_hbm.at[wid], idx_v)
        pltpu.sync_copy(pos_hbm, pos_v)

        def gather(l, b):
            return pltpu.make_async_copy(
                tok_hbm.at[idx_v.at[l, :]], bufs[b], sems[b])

        def write(l, b):
            return pltpu.make_async_copy(
                bufs[b], out_hbm.at[pl.ds(b0, bpw), l, :], sems[b])

        def add_pos(l, b):
            prow = [pos_v[l, pl.ds(k * LANES, LANES)] for k in range(nvr)]

            def row_body(r, c):
                for k in range(nvr):
                    sl = pl.ds(k * LANES, LANES)
                    bufs[b][r, sl] = bufs[b][r, sl] + prow[k]
                return c

            lax.fori_loop(0, bpw, row_body, 0)

        gather(0, 0).start()
        gather(1, 1).start()
        for l in range(4):
            b = l % NBUF
            gather(l, b).wait()
            add_pos(l, b)
            write(l, b).start()
            if l + 2 < 4 + 2:
                bb = (l + 2) % NBUF
                if l - 2 >= 0:
                    write(l - 2, bb).wait()
                gather(l + 2, bb).start()

        def main_body(i, c):
            for db in range(NBUF):
                l = NBUF * i + db
                b = db
                gather(l, b).wait()
                add_pos(l, b)
                write(l, b).start()
                bb = (db + 2) % NBUF
                write(l - 2, bb).wait()
                gather(l + 2, bb).start()
            return c

        lax.fori_loop(1, seq_len // NBUF - 1, main_body, 0)

        for l in range(seq_len - NBUF, seq_len):
            b = l % NBUF
            gather(l, b).wait()
            add_pos(l, b)
            write(l, b).start()
            bb = (b + 2) % NBUF
            if l + 2 < seq_len:
                write(l - 2, bb).wait()
                gather(l + 2, bb).start()
        for l in range(seq_len - 2, seq_len):
            write(l - 2, (l - 2) % NBUF).wait()
        for l in range(seq_len - 2, seq_len):
            write(l, l % NBUF).wait()

    return run


def kernel(seq, token_table, pos_table):
    batch, seq_len = seq.shape
    vocab, depth = token_table.shape
    n_rows = batch * seq_len
    assert batch % NW == 0 and depth % LANES == 0 and seq_len % NBUF == 0

    bpw = batch // NW
    seq_perm = jnp.transpose(
        seq.reshape(NW, bpw, seq_len).astype(jnp.int32), (0, 2, 1))

    return _seq_embed_call(batch, seq_len, depth)(
        seq_perm, token_table, pos_table)
